# baseline (speedup 1.0000x reference)
import functools

import jax
import jax.numpy as jnp
from jax import lax
from jax.experimental import pallas as pl
from jax.experimental.pallas import tpu as pltpu
from jax.experimental.pallas import tpu_sc as plsc

VOCAB = 100000
EMBED_DIM = 128
BATCH = 16384
HIST_LEN = 50

NC = 2
NS = 16
NW = NC * NS
TOTAL_ROWS = BATCH * HIST_LEN
ROWS_PER_W = TOTAL_ROWS // NW
CHUNK = 128
N_CHUNK = ROWS_PER_W // CHUNK


NBUF = 5
NGATHER = 3


def _sc_gather(ids_hbm, table_hbm, out_hbm, idx_v, bufs, gsems, wsems):
    c = lax.axis_index("c")
    s = lax.axis_index("s")
    wid = s * NC + c
    base0 = wid * ROWS_PER_W

    pltpu.sync_copy(ids_hbm.at[wid], idx_v)

    def start_gather(j, b):
        pltpu.async_copy(table_hbm.at[idx_v.at[j]], bufs[b], gsems[b])

    def wait_gather(b):
        pltpu.make_async_copy(table_hbm.at[idx_v.at[0]], bufs[b], gsems[b]).wait()

    def start_write(j, b):
        pltpu.async_copy(bufs[b], out_hbm.at[pl.ds(base0 + j * CHUNK, CHUNK)], wsems[b])

    def wait_write(b):
        pltpu.make_async_copy(bufs[b], out_hbm.at[pl.ds(base0, CHUNK)], wsems[b]).wait()

    for b in range(NGATHER):
        start_gather(b, b)

    @pl.loop(0, N_CHUNK, step=NBUF)
    def _(jj):
        for b in range(NBUF):
            j = jj + b
            wait_gather(b)
            start_write(j, b)
            n = j + NGATHER
            bn = (b + NGATHER) % NBUF

            @pl.when(jnp.logical_and(n < N_CHUNK, n >= NBUF))
            def _():
                wait_write(bn)

            @pl.when(n < N_CHUNK)
            def _():
                start_gather(n, bn)

    for b in range(NBUF):
        wait_write(b)


@jax.jit
def _embed(ids_grouped, item_table):
    mesh = plsc.VectorSubcoreMesh(core_axis_name="c", subcore_axis_name="s")
    run = pl.kernel(
        _sc_gather,
        out_type=jax.ShapeDtypeStruct((TOTAL_ROWS, EMBED_DIM), jnp.float32),
        mesh=mesh,
        scratch_types=[
            pltpu.VMEM((N_CHUNK, CHUNK), jnp.int32),
            tuple(pltpu.VMEM((CHUNK, EMBED_DIM), jnp.float32) for _ in range(NBUF)),
            tuple(pltpu.SemaphoreType.DMA for _ in range(NBUF)),
            tuple(pltpu.SemaphoreType.DMA for _ in range(NBUF)),
        ],
    )
    return run(ids_grouped, item_table)


def kernel(item_ids, item_table):
    ids_grouped = item_ids.astype(jnp.int32).reshape(NW, N_CHUNK, CHUNK)
    out = _embed(ids_grouped, item_table)
    return out.reshape(BATCH, HIST_LEN, EMBED_DIM)


# speedup vs baseline: 3.4571x; 3.4571x over previous
import functools

import jax
import jax.numpy as jnp
from jax import lax
from jax.experimental import pallas as pl
from jax.experimental.pallas import tpu as pltpu
from jax.experimental.pallas import tpu_sc as plsc

VOCAB = 100000
EMBED_DIM = 128
BATCH = 16384
HIST_LEN = 50

NC = 2
NS = 16
NW = NC * NS
TOTAL_ROWS = BATCH * HIST_LEN
ROWS_PER_W = TOTAL_ROWS // NW
CHUNK = 128
N_CHUNK = ROWS_PER_W // CHUNK


NBUF = 5
NGATHER = 3


def _sc_gather(ids_hbm, table_hbm, out_hbm, idx_v, bufs, gsems, wsems):
    c = lax.axis_index("c")
    s = lax.axis_index("s")
    wid = s * NC + c
    base0 = wid * ROWS_PER_W

    pltpu.sync_copy(ids_hbm.at[wid], idx_v)

    def start_gather(j, b):
        pltpu.async_copy(table_hbm.at[idx_v.at[j]], bufs[b], gsems[b])

    def wait_gather(b):
        pltpu.make_async_copy(table_hbm.at[idx_v.at[0]], bufs[b], gsems[b]).wait()

    def start_write(j, b):
        pltpu.async_copy(bufs[b], out_hbm.at[pl.ds(base0 + j * CHUNK, CHUNK)], wsems[b])

    def wait_write(b):
        pltpu.make_async_copy(bufs[b], out_hbm.at[pl.ds(base0, CHUNK)], wsems[b]).wait()

    for b in range(NGATHER):
        start_gather(b, b)

    @pl.loop(0, N_CHUNK, step=NBUF)
    def _(jj):
        for b in range(NBUF):
            j = jj + b
            wait_gather(b)
            start_write(j, b)
            n = j + NGATHER
            bn = (b + NGATHER) % NBUF

            @pl.when(jnp.logical_and(n < N_CHUNK, n >= NBUF))
            def _():
                wait_write(bn)

            @pl.when(n < N_CHUNK)
            def _():
                start_gather(n, bn)

    for b in range(NBUF):
        wait_write(b)


@jax.jit
def _embed(ids_grouped, item_table):
    mesh = plsc.VectorSubcoreMesh(core_axis_name="c", subcore_axis_name="s")
    run = pl.kernel(
        _sc_gather,
        out_type=jax.ShapeDtypeStruct((TOTAL_ROWS, EMBED_DIM), jnp.float32),
        mesh=mesh,
        compiler_params=pltpu.CompilerParams(use_tc_tiling_on_sc=True),
        scratch_types=[
            pltpu.VMEM((N_CHUNK, CHUNK), jnp.int32),
            tuple(pltpu.VMEM((CHUNK, EMBED_DIM), jnp.float32) for _ in range(NBUF)),
            tuple(pltpu.SemaphoreType.DMA for _ in range(NBUF)),
            tuple(pltpu.SemaphoreType.DMA for _ in range(NBUF)),
        ],
    )
    return run(ids_grouped, item_table)


def kernel(item_ids, item_table):
    ids_t = item_ids.astype(jnp.int32).T
    ids_grouped = ids_t.reshape(NW, N_CHUNK, CHUNK)
    out = _embed(ids_grouped, item_table)
    return out.reshape(HIST_LEN, BATCH, EMBED_DIM).transpose(1, 0, 2)
